# baseline (speedup 1.0000x reference)
import functools

import jax
import jax.numpy as jnp
from jax import lax
from jax.experimental import pallas as pl
from jax.experimental.pallas import tpu as pltpu
from jax.experimental.pallas import tpu_sc as plsc

_V = 1000000
_B = 16384
_D = 16
_NC = 2
_NS = 16
_NW = _NC * _NS
_BW = _B // _NW
_NCHUNK = 4
_CB = _BW // _NCHUNK
_L = 16

_mesh = plsc.VectorSubcoreMesh(core_axis_name="c", subcore_axis_name="s")


@functools.partial(
    pl.kernel,
    out_type=jax.ShapeDtypeStruct(((_D + 1) * _B,), jnp.float32),
    mesh=_mesh,
    scratch_types=[
        pltpu.VMEM((_BW,), jnp.int32),
        pltpu.VMEM((_D * _BW,), jnp.int32),
        pltpu.VMEM(((_D + 1) * _BW,), jnp.float32),
        pltpu.SemaphoreType.DMA,
        pltpu.SemaphoreType.DMA,
    ],
)
def _emb_kernel(idx_hbm, pos_hbm, learn_hbm, out_hbm,
                idx_v, offs_v, emb_v, gsem, osem):
    wid = lax.axis_index("s") * _NC + lax.axis_index("c")
    base = wid * _BW

    pltpu.sync_copy(idx_hbm.at[pl.ds(base, _BW)], idx_v)

    copies = []
    for j in range(_NCHUNK):
        copies.append(pltpu.async_copy(
            learn_hbm.at[idx_v.at[pl.ds(j * _CB, _CB)]],
            emb_v.at[pl.ds(_D * _BW + j * _CB, _CB)], gsem))

    def offset_body(k, carry):
        v = idx_v[pl.ds(k * _L, _L)]
        for f in range(_D):
            offs_v[pl.ds(f * _BW + k * _L, _L)] = v + f * _V
        return carry
    lax.fori_loop(0, _BW // _L, offset_body, 0)

    for f in range(_D):
        for j in range(_NCHUNK):
            o = f * _BW + j * _CB
            copies.append(pltpu.async_copy(
                pos_hbm.at[offs_v.at[pl.ds(o, _CB)]],
                emb_v.at[pl.ds(o, _CB)], gsem))
    for cp in copies:
        cp.wait()

    outs = []
    for f in range(_D + 1):
        outs.append(pltpu.async_copy(
            emb_v.at[pl.ds(f * _BW, _BW)],
            out_hbm.at[pl.ds(f * _B + base, _BW)], osem))
    for cp in outs:
        cp.wait()


def kernel(indices, pos_table, learn_table):
    idx = indices.astype(jnp.int32)
    pos_flat = pos_table.T.reshape(-1)
    learn_flat = learn_table.reshape(-1)
    out_flat = _emb_kernel(idx, pos_flat, learn_flat)
    return out_flat.reshape(_D + 1, _B).T


# speedup vs baseline: 2.8266x; 2.8266x over previous
import functools

import jax
import jax.numpy as jnp
from jax import lax
from jax.experimental import pallas as pl
from jax.experimental.pallas import tpu as pltpu
from jax.experimental.pallas import tpu_sc as plsc

_B = 16384
_D = 16
_NC = 2
_NS = 16
_NW = _NC * _NS
_BW = _B // _NW
_NCHUNK = 4
_CB = _BW // _NCHUNK

_mesh = plsc.VectorSubcoreMesh(core_axis_name="c", subcore_axis_name="s")


@functools.partial(
    pl.kernel,
    out_type=(
        jax.ShapeDtypeStruct((_B, _D), jnp.float32),
        jax.ShapeDtypeStruct((_B,), jnp.float32),
    ),
    mesh=_mesh,
    scratch_types=[
        pltpu.VMEM((_BW,), jnp.int32),
        pltpu.VMEM((_BW, _D), jnp.float32),
        pltpu.VMEM((_BW,), jnp.float32),
        pltpu.SemaphoreType.DMA,
        pltpu.SemaphoreType.DMA,
    ],
    compiler_params=pltpu.CompilerParams(use_tc_tiling_on_sc=False),
)
def _emb_kernel(idx_hbm, pos_hbm, learn_hbm, pos_out, learn_out,
                idx_v, pos_v, learn_v, psem, lsem):
    wid = lax.axis_index("s") * _NC + lax.axis_index("c")
    base = wid * _BW

    pltpu.sync_copy(idx_hbm.at[pl.ds(base, _BW)], idx_v)

    copies = []
    for j in range(_NCHUNK):
        idx_j = idx_v.at[pl.ds(j * _CB, _CB)]
        copies.append(pltpu.async_copy(
            pos_hbm.at[idx_j], pos_v.at[pl.ds(j * _CB, _CB)], psem))
        copies.append(pltpu.async_copy(
            learn_hbm.at[idx_j], learn_v.at[pl.ds(j * _CB, _CB)], lsem))
    for cp in copies:
        cp.wait()

    pltpu.sync_copy(pos_v, pos_out.at[pl.ds(base, _BW)])
    pltpu.sync_copy(learn_v, learn_out.at[pl.ds(base, _BW)])


def kernel(indices, pos_table, learn_table):
    idx = indices.astype(jnp.int32)
    learn_flat = learn_table.reshape(-1)
    pos_emb, learn_emb = _emb_kernel(idx, pos_table, learn_flat)
    return jnp.concatenate([pos_emb, learn_emb[:, None]], axis=-1)


# speedup vs baseline: 2.8608x; 1.0121x over previous
import functools

import jax
import jax.numpy as jnp
from jax import lax
from jax.experimental import pallas as pl
from jax.experimental.pallas import tpu as pltpu
from jax.experimental.pallas import tpu_sc as plsc

_B = 16384
_D = 16
_NC = 2
_NS = 16
_NW = _NC * _NS
_BW = _B // _NW
_NCHUNK = 4
_CB = _BW // _NCHUNK
_L = 16

_mesh = plsc.VectorSubcoreMesh(core_axis_name="c", subcore_axis_name="s")


@functools.partial(
    pl.kernel,
    out_type=jax.ShapeDtypeStruct((_D + 1, _B), jnp.float32),
    mesh=_mesh,
    scratch_types=[
        pltpu.VMEM((_BW,), jnp.int32),
        pltpu.VMEM((_BW, _D), jnp.float32),
        pltpu.VMEM((_BW * (_D + 1),), jnp.float32),
        pltpu.SemaphoreType.DMA,
        pltpu.SemaphoreType.DMA,
    ],
    compiler_params=pltpu.CompilerParams(
        use_tc_tiling_on_sc=False, needs_layout_passes=False),
)
def _emb_kernel(idx_hbm, pos_hbm, learn_hbm, out_hbm,
                idx_v, rows_v, emb_v, psem, lsem):
    wid = lax.axis_index("s") * _NC + lax.axis_index("c")
    base = wid * _BW

    pltpu.sync_copy(idx_hbm.at[pl.ds(base, _BW)], idx_v)

    copies = []
    for j in range(_NCHUNK):
        idx_j = idx_v.at[pl.ds(j * _CB, _CB)]
        copies.append(pltpu.async_copy(
            pos_hbm.at[idx_j],
            rows_v.at[pl.ds(j * _CB, _CB)], psem))
        copies.append(pltpu.async_copy(
            learn_hbm.at[idx_j],
            emb_v.at[pl.ds(_D * _BW + j * _CB, _CB)], lsem))
    for cp in copies:
        cp.wait()

    lanes = lax.iota(jnp.int32, _L)
    def transpose_body(k, carry):
        rows16 = lanes + k * _L
        for f in range(_D):
            emb_v[pl.ds(f * _BW + k * _L, _L)] = plsc.load_gather(
                rows_v, [rows16, jnp.full((_L,), f, jnp.int32)])
        return carry
    lax.fori_loop(0, _BW // _L, transpose_body, 0)

    outs = []
    for f in range(_D + 1):
        outs.append(pltpu.async_copy(
            emb_v.at[pl.ds(f * _BW, _BW)],
            out_hbm.at[f].at[pl.ds(base, _BW)], psem))
    for cp in outs:
        cp.wait()


def kernel(indices, pos_table, learn_table):
    idx = indices.astype(jnp.int32)
    learn_flat = learn_table.reshape(-1)
    out_t = _emb_kernel(idx, pos_table, learn_flat)
    return out_t.T


# speedup vs baseline: 10.0911x; 3.5274x over previous
import functools

import jax
import jax.numpy as jnp
from jax import lax
from jax.experimental import pallas as pl
from jax.experimental.pallas import tpu as pltpu
from jax.experimental.pallas import tpu_sc as plsc

_B = 16384
_D = 16
_NC = 2
_NS = 16
_NW = _NC * _NS
_BW = _B // _NW
_NCHUNK = 4
_CB = _BW // _NCHUNK
_L = 16
_NBUF = 16

_mesh = plsc.VectorSubcoreMesh(core_axis_name="c", subcore_axis_name="s")


@functools.partial(
    pl.kernel,
    out_type=jax.ShapeDtypeStruct(((_D + 1) * _B,), jnp.float32),
    mesh=_mesh,
    scratch_types=[
        pltpu.VMEM((_BW,), jnp.int32),
        pltpu.VMEM((_NBUF, _D, 128), jnp.float32),
        pltpu.VMEM(((_D + 1) * _BW,), jnp.float32),
        pltpu.SemaphoreType.DMA,
        pltpu.SemaphoreType.DMA,
        pltpu.SemaphoreType.DMA,
    ],
    compiler_params=pltpu.CompilerParams(needs_layout_passes=False),
)
def _emb_kernel(idx_hbm, pos_hbm, learn_hbm, out_hbm,
                idx_v, win_v, emb_v, wsem, lsem, osem):
    wid = lax.axis_index("s") * _NC + lax.axis_index("c")
    base = wid * _BW

    pltpu.sync_copy(idx_hbm.at[pl.ds(base, _BW)], idx_v)

    lcopies = []
    for j in range(_NCHUNK):
        lcopies.append(pltpu.async_copy(
            learn_hbm.at[idx_v.at[pl.ds(j * _CB, _CB)]],
            emb_v.at[pl.ds(_D * _BW + j * _CB, _CB)], lsem))

    lanes = lax.iota(jnp.int32, _L)
    outpos = lanes * _BW

    def group_body(k, carry):
        vv = idx_v[pl.ds(k * _L, _L)]
        for u in range(_NBUF):
            cb = (vv[u] >> 7) * 128
            pltpu.async_copy(
                pos_hbm.at[:, pl.ds(cb, 128)], win_v.at[u], wsem)
        for u in range(_NBUF):
            pltpu.make_async_copy(
                pos_hbm.at[:, pl.ds(0, 128)], win_v.at[u], wsem).wait()
            col = vv[u] & 127
            vals = plsc.load_gather(
                win_v.at[u], [lanes, jnp.full((_L,), col, jnp.int32)])
            plsc.store_scatter(emb_v, [outpos + (k * _L + u)], vals)
        return carry

    lax.fori_loop(0, _BW // _NBUF, group_body, 0)

    for cp in lcopies:
        cp.wait()

    outs = []
    for f in range(_D + 1):
        outs.append(pltpu.async_copy(
            emb_v.at[pl.ds(f * _BW, _BW)],
            out_hbm.at[pl.ds(f * _B + base, _BW)], osem))
    for cp in outs:
        cp.wait()


def kernel(indices, pos_table, learn_table):
    idx = indices.astype(jnp.int32)
    pos_t = pos_table.T
    learn_flat = learn_table.reshape(-1)
    out_flat = _emb_kernel(idx, pos_t, learn_flat)
    return out_flat.reshape(_D + 1, _B).T
